# baseline (speedup 1.0000x reference)
import functools

import jax
import jax.numpy as jnp
from jax import lax
from jax.experimental import pallas as pl
from jax.experimental.pallas import tpu as pltpu
from jax.experimental.pallas import tpu_sc as plsc

D = 32
_IDX_CHUNK = 128

try:
    _info = plsc.get_sparse_core_info()
    _NC, _NS = _info.num_cores, _info.num_subcores
except Exception:
    _NC, _NS = 2, 16
_NW = _NC * _NS


def _make_sc_gather(B):
    bpw = B // _NW
    nchunks = bpw // _IDX_CHUNK
    mesh = plsc.VectorSubcoreMesh(core_axis_name="c", subcore_axis_name="s")

    @functools.partial(
        pl.kernel,
        mesh=mesh,
        out_type=[jax.ShapeDtypeStruct((B, D), jnp.float32)] * 4,
        scratch_types=[
            pltpu.VMEM((bpw,), jnp.int32),
            pltpu.VMEM((bpw,), jnp.int32),
            pltpu.VMEM((bpw, D), jnp.float32),
            pltpu.VMEM((bpw, D), jnp.float32),
            pltpu.VMEM((bpw, D), jnp.float32),
            pltpu.VMEM((bpw, D), jnp.float32),
            pltpu.SemaphoreType.DMA,
        ],
        compiler_params=pltpu.CompilerParams(use_tc_tiling_on_sc=False),
    )
    def sc_gather(uid_hbm, iid_hbm, gu_hbm, gi_hbm, mu_hbm, mi_hbm,
                  gu_out, gi_out, mu_out, mi_out,
                  uidx, iidx, gu, gi, mu, mi, sem):
        wid = lax.axis_index("s") * _NC + lax.axis_index("c")
        base = wid * bpw
        pltpu.sync_copy(uid_hbm.at[pl.ds(base, bpw)], uidx)
        pltpu.sync_copy(iid_hbm.at[pl.ds(base, bpw)], iidx)
        copies = []
        for j in range(nchunks):
            s = pl.ds(j * _IDX_CHUNK, _IDX_CHUNK)
            copies.append(pltpu.async_copy(gu_hbm.at[uidx.at[s]], gu.at[s], sem))
            copies.append(pltpu.async_copy(gi_hbm.at[iidx.at[s]], gi.at[s], sem))
            copies.append(pltpu.async_copy(mu_hbm.at[uidx.at[s]], mu.at[s], sem))
            copies.append(pltpu.async_copy(mi_hbm.at[iidx.at[s]], mi.at[s], sem))
        for c in copies:
            c.wait()
        pltpu.sync_copy(gu, gu_out.at[pl.ds(base, bpw)])
        pltpu.sync_copy(gi, gi_out.at[pl.ds(base, bpw)])
        pltpu.sync_copy(mu, mu_out.at[pl.ds(base, bpw)])
        pltpu.sync_copy(mi, mi_out.at[pl.ds(base, bpw)])

    return sc_gather


def _tc_body(gu_ref, gi_ref, mu_ref, mi_ref, fcw_ref, fcb_ref,
             w1u_ref, w1i_ref, b1_ref, w2_ref, b2_ref, w3_ref, b3_ref,
             owg_ref, owh_ref, ob_ref, out_ref):
    f32 = jnp.float32
    gmf = gu_ref[...] * gi_ref[...]
    gmf = jnp.dot(gmf, fcw_ref[...], preferred_element_type=f32)
    gmf = gmf + fcb_ref[...][None, :]
    h = jnp.dot(mu_ref[...], w1u_ref[...], preferred_element_type=f32)
    h = h + jnp.dot(mi_ref[...], w1i_ref[...], preferred_element_type=f32)
    h = jnp.maximum(h + b1_ref[...][None, :], 0.0)
    h = jnp.dot(h, w2_ref[...], preferred_element_type=f32)
    h = jnp.maximum(h + b2_ref[...][None, :], 0.0)
    h = jnp.dot(h, w3_ref[...], preferred_element_type=f32)
    h = jnp.maximum(h + b3_ref[...][None, :], 0.0)
    out = jnp.sum(gmf * owg_ref[...][None, :], axis=1)
    out = out + jnp.sum(h * owh_ref[...][None, :], axis=1)
    out_ref[...] = out + ob_ref[...]


def _tc_dense(gu, gi, mu, mi, gmf_fc_w, gmf_fc_b, w1, b1, w2, b2, w3, b3,
              out_w, out_b):
    B = gu.shape[0]
    BM = min(2048, B)
    w1u = w1[:D]
    w1i = w1[D:]
    owg = out_w[:D, 0]
    owh = out_w[D:, 0]
    full = lambda shape: pl.BlockSpec(shape, lambda i: (0,) * len(shape))
    grid = (B // BM,)
    return pl.pallas_call(
        _tc_body,
        grid=grid,
        in_specs=[
            pl.BlockSpec((BM, D), lambda i: (i, 0)),
            pl.BlockSpec((BM, D), lambda i: (i, 0)),
            pl.BlockSpec((BM, D), lambda i: (i, 0)),
            pl.BlockSpec((BM, D), lambda i: (i, 0)),
            full((D, D)), full((D,)),
            full((D, 64)), full((D, 64)), full((64,)),
            full((64, 32)), full((32,)),
            full((32, 16)), full((16,)),
            full((D,)), full((16,)), full((1,)),
        ],
        out_specs=pl.BlockSpec((BM,), lambda i: (i,)),
        out_shape=jax.ShapeDtypeStruct((B,), jnp.float32),
        compiler_params=pltpu.CompilerParams(
            dimension_semantics=("parallel",)),
    )(gu, gi, mu, mi, gmf_fc_w, gmf_fc_b, w1u, w1i, b1, w2, b2, w3, b3,
      owg, owh, out_b)


def kernel(user_id, item_id, gmf_user_emb, gmf_item_emb, mlp_user_emb,
           mlp_item_emb, gmf_fc_w, gmf_fc_b, w1, b1, w2, b2, w3, b3,
           out_w, out_b):
    B = user_id.shape[0]
    uid = user_id.astype(jnp.int32)
    iid = item_id.astype(jnp.int32)
    gu, gi, mu, mi = _make_sc_gather(B)(
        uid, iid, gmf_user_emb, gmf_item_emb, mlp_user_emb, mlp_item_emb)
    return _tc_dense(gu, gi, mu, mi, gmf_fc_w, gmf_fc_b, w1, b1, w2, b2,
                     w3, b3, out_w, out_b)


# speedup vs baseline: 3.4901x; 3.4901x over previous
import functools

import jax
import jax.numpy as jnp
from jax import lax
from jax.experimental import pallas as pl
from jax.experimental.pallas import tpu as pltpu
from jax.experimental.pallas import tpu_sc as plsc

D = 32
G = 16

try:
    _info = plsc.get_sparse_core_info()
    _NC, _NS = _info.num_cores, _info.num_subcores
except Exception:
    _NC, _NS = 2, 16
_NW = _NC * _NS


def _make_sc_gather(B):
    bpw = B // _NW
    mesh = plsc.VectorSubcoreMesh(core_axis_name="c", subcore_axis_name="s")

    @functools.partial(
        pl.kernel,
        mesh=mesh,
        out_type=[jax.ShapeDtypeStruct((D, B), jnp.float32)] * 4,
        scratch_types=[
            pltpu.VMEM((bpw,), jnp.int32),
            pltpu.VMEM((bpw,), jnp.int32),
            pltpu.VMEM((G, D, 128), jnp.float32),
            pltpu.VMEM((D, bpw), jnp.float32),
            pltpu.SemaphoreType.DMA,
        ],
        compiler_params=pltpu.CompilerParams(
            use_tc_tiling_on_sc=True,
            disable_bounds_checks=True,
            needs_layout_passes=False,
        ),
    )
    def sc_gather(uid_hbm, iid_hbm, gu_hbm, gi_hbm, mu_hbm, mi_hbm,
                  gu_out, gi_out, mu_out, mi_out,
                  uidx_v, iidx_v, slab, stag, sem):
        wid = lax.axis_index("s") * _NC + lax.axis_index("c")
        base = wid * bpw
        pltpu.sync_copy(uid_hbm.at[pl.ds(base, bpw)], uidx_v)
        pltpu.sync_copy(iid_hbm.at[pl.ds(base, bpw)], iidx_v)

        iota16 = lax.iota(jnp.int32, G)

        def gather_table(tab, idx_v, out):
            def group_body(g, carry):
                idx16 = idx_v[pl.ds(g * G, G)]
                blk16 = (idx16 >> 7) * 128
                for k in range(G):
                    blk = pl.multiple_of(blk16[k], 128)
                    pltpu.async_copy(tab.at[:, pl.ds(blk, 128)],
                                     slab.at[k], sem)
                for k in range(G):
                    pltpu.make_async_copy(tab.at[:, pl.ds(0, 128)],
                                          slab.at[k], sem).wait()
                lanes = idx16 & 127
                spos = g * G + iota16
                for c in range(D):
                    cvec = jnp.full((G,), c, jnp.int32)
                    vals = plsc.load_gather(slab, [iota16, cvec, lanes])
                    plsc.store_scatter(stag, [cvec, spos], vals)
                return carry

            lax.fori_loop(0, bpw // G, group_body, 0)
            pltpu.sync_copy(stag, out.at[:, pl.ds(base, bpw)])

        gather_table(gu_hbm, uidx_v, gu_out)
        gather_table(mu_hbm, uidx_v, mu_out)
        gather_table(gi_hbm, iidx_v, gi_out)
        gather_table(mi_hbm, iidx_v, mi_out)

    return sc_gather


def _tc_body(gu_ref, gi_ref, mu_ref, mi_ref, fcw_ref, fcb_ref,
             w1u_ref, w1i_ref, b1_ref, w2_ref, b2_ref, w3_ref, b3_ref,
             owg_ref, owh_ref, ob_ref, out_ref):
    f32 = jnp.float32
    dn = (((0,), (0,)), ((), ()))
    gmf_t = gu_ref[...] * gi_ref[...]
    gmf = lax.dot_general(gmf_t, fcw_ref[...], dn, preferred_element_type=f32)
    gmf = gmf + fcb_ref[...][None, :]
    h = lax.dot_general(mu_ref[...], w1u_ref[...], dn,
                        preferred_element_type=f32)
    h = h + lax.dot_general(mi_ref[...], w1i_ref[...], dn,
                            preferred_element_type=f32)
    h = jnp.maximum(h + b1_ref[...][None, :], 0.0)
    h = jnp.dot(h, w2_ref[...], preferred_element_type=f32)
    h = jnp.maximum(h + b2_ref[...][None, :], 0.0)
    h = jnp.dot(h, w3_ref[...], preferred_element_type=f32)
    h = jnp.maximum(h + b3_ref[...][None, :], 0.0)
    out = jnp.sum(gmf * owg_ref[...][None, :], axis=1)
    out = out + jnp.sum(h * owh_ref[...][None, :], axis=1)
    out_ref[...] = out + ob_ref[...]


def _tc_dense(gu_t, gi_t, mu_t, mi_t, gmf_fc_w, gmf_fc_b, w1, b1, w2, b2,
              w3, b3, out_w, out_b):
    B = gu_t.shape[1]
    BM = min(2048, B)
    w1u = w1[:D]
    w1i = w1[D:]
    owg = out_w[:D, 0]
    owh = out_w[D:, 0]
    full = lambda shape: pl.BlockSpec(shape, lambda i: (0,) * len(shape))
    grid = (B // BM,)
    act = pl.BlockSpec((D, BM), lambda i: (0, i))
    return pl.pallas_call(
        _tc_body,
        grid=grid,
        in_specs=[
            act, act, act, act,
            full((D, D)), full((D,)),
            full((D, 64)), full((D, 64)), full((64,)),
            full((64, 32)), full((32,)),
            full((32, 16)), full((16,)),
            full((D,)), full((16,)), full((1,)),
        ],
        out_specs=pl.BlockSpec((BM,), lambda i: (i,)),
        out_shape=jax.ShapeDtypeStruct((B,), jnp.float32),
        compiler_params=pltpu.CompilerParams(
            dimension_semantics=("parallel",)),
    )(gu_t, gi_t, mu_t, mi_t, gmf_fc_w, gmf_fc_b, w1u, w1i, b1, w2, b2,
      w3, b3, owg, owh, out_b)


def kernel(user_id, item_id, gmf_user_emb, gmf_item_emb, mlp_user_emb,
           mlp_item_emb, gmf_fc_w, gmf_fc_b, w1, b1, w2, b2, w3, b3,
           out_w, out_b):
    B = user_id.shape[0]
    uid = user_id.astype(jnp.int32)
    iid = item_id.astype(jnp.int32)
    gu_t, gi_t, mu_t, mi_t = _make_sc_gather(B)(
        uid, iid, gmf_user_emb.T, gmf_item_emb.T, mlp_user_emb.T,
        mlp_item_emb.T)
    return _tc_dense(gu_t, gi_t, mu_t, mi_t, gmf_fc_w, gmf_fc_b, w1, b1,
                     w2, b2, w3, b3, out_w, out_b)


# speedup vs baseline: 4.2919x; 1.2297x over previous
import functools

import jax
import jax.numpy as jnp
from jax import lax
from jax.experimental import pallas as pl
from jax.experimental.pallas import tpu as pltpu
from jax.experimental.pallas import tpu_sc as plsc

D = 32
G = 16
R = 24

try:
    _info = plsc.get_sparse_core_info()
    _NC, _NS = _info.num_cores, _info.num_subcores
except Exception:
    _NC, _NS = 2, 16
_NW = _NC * _NS


def _make_sc_gather(B):
    bpw = B // _NW
    mesh = plsc.VectorSubcoreMesh(core_axis_name="c", subcore_axis_name="s")

    @functools.partial(
        pl.kernel,
        mesh=mesh,
        out_type=[jax.ShapeDtypeStruct((D, B), jnp.float32)] * 4,
        scratch_types=[
            pltpu.VMEM((bpw,), jnp.int32),
            pltpu.VMEM((bpw,), jnp.int32),
            pltpu.VMEM((R, D, 128), jnp.float32),
            pltpu.VMEM((D, bpw), jnp.float32),
            pltpu.SemaphoreType.DMA,
        ],
        compiler_params=pltpu.CompilerParams(
            use_tc_tiling_on_sc=True,
            disable_bounds_checks=True,
            needs_layout_passes=False,
        ),
    )
    def sc_gather(uid_hbm, iid_hbm, gu_hbm, gi_hbm, mu_hbm, mi_hbm,
                  gu_out, gi_out, mu_out, mi_out,
                  uidx_v, iidx_v, slab, stag, sem):
        wid = lax.axis_index("s") * _NC + lax.axis_index("c")
        base = wid * bpw
        pltpu.sync_copy(uid_hbm.at[pl.ds(base, bpw)], uidx_v)
        pltpu.sync_copy(iid_hbm.at[pl.ds(base, bpw)], iidx_v)

        iota16 = lax.iota(jnp.int32, G)
        ngroups = bpw // G

        def gather_table(tab, idx_v, out):
            def issue_half(gq, klo):
                @pl.when(gq * G + klo < bpw)
                def _():
                    idx16 = idx_v[pl.ds(gq * G, G)]
                    for k in range(klo, klo + 8):
                        blk = pl.multiple_of((idx16[k] >> 7) * 128, 128)
                        slot = lax.rem(gq * G + k, R)
                        pltpu.async_copy(tab.at[:, pl.ds(blk, 128)],
                                         slab.at[slot], sem)

            issue_half(0, 0)
            issue_half(0, 8)
            issue_half(1, 0)

            def group_body(g, carry):
                for _ in range(G):
                    pltpu.make_async_copy(tab.at[:, pl.ds(0, 128)],
                                          slab.at[0], sem).wait()
                idx16 = idx_v[pl.ds(g * G, G)]
                lanes = idx16 & 127
                bvec = lax.rem(g * G + iota16, R)
                spos = g * G + iota16
                for c in range(D):
                    cvec = jnp.full((G,), c, jnp.int32)
                    vals = plsc.load_gather(slab, [bvec, cvec, lanes])
                    plsc.store_scatter(stag, [cvec, spos], vals)
                issue_half(g + 1, 8)
                issue_half(g + 2, 0)
                return carry

            lax.fori_loop(0, ngroups, group_body, 0)
            pltpu.sync_copy(stag, out.at[:, pl.ds(base, bpw)])

        gather_table(gu_hbm, uidx_v, gu_out)
        gather_table(mu_hbm, uidx_v, mu_out)
        gather_table(gi_hbm, iidx_v, gi_out)
        gather_table(mi_hbm, iidx_v, mi_out)

    return sc_gather


def _tc_body(gu_ref, gi_ref, mu_ref, mi_ref, fcw_ref, fcb_ref,
             w1u_ref, w1i_ref, b1_ref, w2_ref, b2_ref, w3_ref, b3_ref,
             owg_ref, owh_ref, ob_ref, out_ref):
    f32 = jnp.float32
    dn = (((0,), (0,)), ((), ()))
    gmf_t = gu_ref[...] * gi_ref[...]
    gmf = lax.dot_general(gmf_t, fcw_ref[...], dn, preferred_element_type=f32)
    gmf = gmf + fcb_ref[...][None, :]
    h = lax.dot_general(mu_ref[...], w1u_ref[...], dn,
                        preferred_element_type=f32)
    h = h + lax.dot_general(mi_ref[...], w1i_ref[...], dn,
                            preferred_element_type=f32)
    h = jnp.maximum(h + b1_ref[...][None, :], 0.0)
    h = jnp.dot(h, w2_ref[...], preferred_element_type=f32)
    h = jnp.maximum(h + b2_ref[...][None, :], 0.0)
    h = jnp.dot(h, w3_ref[...], preferred_element_type=f32)
    h = jnp.maximum(h + b3_ref[...][None, :], 0.0)
    out = jnp.sum(gmf * owg_ref[...][None, :], axis=1)
    out = out + jnp.sum(h * owh_ref[...][None, :], axis=1)
    out_ref[...] = out + ob_ref[...]


def _tc_dense(gu_t, gi_t, mu_t, mi_t, gmf_fc_w, gmf_fc_b, w1, b1, w2, b2,
              w3, b3, out_w, out_b):
    B = gu_t.shape[1]
    BM = min(2048, B)
    w1u = w1[:D]
    w1i = w1[D:]
    owg = out_w[:D, 0]
    owh = out_w[D:, 0]
    full = lambda shape: pl.BlockSpec(shape, lambda i: (0,) * len(shape))
    grid = (B // BM,)
    act = pl.BlockSpec((D, BM), lambda i: (0, i))
    return pl.pallas_call(
        _tc_body,
        grid=grid,
        in_specs=[
            act, act, act, act,
            full((D, D)), full((D,)),
            full((D, 64)), full((D, 64)), full((64,)),
            full((64, 32)), full((32,)),
            full((32, 16)), full((16,)),
            full((D,)), full((16,)), full((1,)),
        ],
        out_specs=pl.BlockSpec((BM,), lambda i: (i,)),
        out_shape=jax.ShapeDtypeStruct((B,), jnp.float32),
        compiler_params=pltpu.CompilerParams(
            dimension_semantics=("parallel",)),
    )(gu_t, gi_t, mu_t, mi_t, gmf_fc_w, gmf_fc_b, w1u, w1i, b1, w2, b2,
      w3, b3, owg, owh, out_b)


def kernel(user_id, item_id, gmf_user_emb, gmf_item_emb, mlp_user_emb,
           mlp_item_emb, gmf_fc_w, gmf_fc_b, w1, b1, w2, b2, w3, b3,
           out_w, out_b):
    B = user_id.shape[0]
    uid = user_id.astype(jnp.int32)
    iid = item_id.astype(jnp.int32)
    gu_t, gi_t, mu_t, mi_t = _make_sc_gather(B)(
        uid, iid, gmf_user_emb.T, gmf_item_emb.T, mlp_user_emb.T,
        mlp_item_emb.T)
    return _tc_dense(gu_t, gi_t, mu_t, mi_t, gmf_fc_w, gmf_fc_b, w1, b1,
                     w2, b2, w3, b3, out_w, out_b)
